# baseline (speedup 1.0000x reference)
import functools

import jax
import jax.numpy as jnp
from jax import lax
from jax.experimental import pallas as pl
from jax.experimental.pallas import tpu as pltpu
from jax.experimental.pallas import tpu_sc as plsc

N = 10000
D = 128
E = 320000
LAYERS = 3

CHUNK = 128
W = 32
MAXCH = 80
NCHP = W * MAXCH
PAD = NCHP * CHUNK - E
NROWS = N + 8
RPT = 640
TAILR = N - 15 * RPT

_mesh = plsc.VectorSubcoreMesh(core_axis_name="c", subcore_axis_name="s")


@functools.partial(
    pl.kernel,
    out_type=[jax.ShapeDtypeStruct((N,), jnp.float32),
              jax.ShapeDtypeStruct((N,), jnp.float32)],
    mesh=_mesh,
    scratch_types=[
        pltpu.VMEM_SHARED((NROWS,), jnp.float32),
        pltpu.VMEM((MAXCH, CHUNK), jnp.int32),
        pltpu.VMEM((CHUNK,), jnp.float32),
        pltpu.VMEM((640,), jnp.float32),
    ],
)
def _deg_kernel(e2d, dout0, dout1, acc, tidx, ones, z):
    c = lax.axis_index("c")
    s = lax.axis_index("s")
    start = (c * 16 + s) * MAXCH
    for i in range(40):
        z[pl.ds(i * 16, 16)] = jnp.zeros((16,), jnp.float32)
    for i in range(8):
        ones[pl.ds(i * 16, 16)] = jnp.ones((16,), jnp.float32)
    zbase = pl.multiple_of(s * 624, 8)
    pltpu.sync_copy(z, acc.at[pl.ds(zbase, 640)])
    plsc.subcore_barrier()
    pltpu.sync_copy(e2d.at[1, pl.ds(start, MAXCH)], tidx)

    @pl.loop(0, MAXCH)
    def _scatter(j):
        pltpu.sync_copy(ones, acc.at[tidx.at[j]], add=True)

    plsc.subcore_barrier()

    for ci, dref in ((0, dout0), (1, dout1)):
        @pl.when((c == ci) & (s < 15))
        def _dump_main(dref=dref):
            o = pl.multiple_of(s * 624, 8)
            pltpu.sync_copy(acc.at[pl.ds(o, 624)], z.at[pl.ds(0, 624)])
            pltpu.sync_copy(z.at[pl.ds(0, 624)], dref.at[pl.ds(o, 624)])

        @pl.when((c == ci) & (s == 15))
        def _dump_tail(dref=dref):
            pltpu.sync_copy(acc.at[pl.ds(9360, 640)], z)
            pltpu.sync_copy(z, dref.at[pl.ds(9360, 640)])


@functools.partial(
    pl.kernel,
    out_type=jax.ShapeDtypeStruct((2, N, D), jnp.float32),
    mesh=_mesh,
    scratch_types=[
        pltpu.VMEM_SHARED((NROWS, D), jnp.float32),
        pltpu.VMEM((MAXCH // 2, CHUNK), jnp.int32),
        pltpu.VMEM((MAXCH // 2, CHUNK), jnp.int32),
        pltpu.VMEM((2, CHUNK, D), jnp.float32),
        pltpu.SemaphoreType.DMA,
        pltpu.SemaphoreType.DMA,
        pltpu.SemaphoreType.DMA,
    ],
)
def _hop_kernel(e2d, u, out, acc, fidx, tidx, rows2, semg, sems0, sems1):
    rows = rows2.at[0]
    sems = (sems0, sems1)
    c = lax.axis_index("c")
    s = lax.axis_index("s")
    start = (c * 16 + s) * MAXCH

    @pl.loop(0, CHUNK)
    def _zrow(r):
        for k in range(D // 16):
            rows[r, pl.ds(k * 16, 16)] = jnp.zeros((16,), jnp.float32)

    rbase = s * RPT

    @pl.when(s < 15)
    def _zero_main():
        for b in range(RPT // CHUNK):
            pltpu.sync_copy(rows, acc.at[pl.ds(rbase + b * CHUNK, CHUNK)])

    @pl.when(s == 15)
    def _zero_tail():
        for b in range(TAILR // CHUNK):
            pltpu.sync_copy(rows, acc.at[pl.ds(rbase + b * CHUNK, CHUNK)])
        pltpu.sync_copy(rows.at[pl.ds(0, TAILR % CHUNK)],
                        acc.at[pl.ds(rbase + (TAILR // CHUNK) * CHUNK,
                                     TAILR % CHUNK)])

    plsc.subcore_barrier()

    def _gs(j, b):
        return pltpu.async_copy(u.at[fidx.at[j]], rows2.at[b], semg)

    def _gw(j, b):
        pltpu.make_async_copy(u.at[fidx.at[j]], rows2.at[b], semg).wait()

    def _ss(j, b):
        pltpu.async_copy(rows2.at[b], acc.at[tidx.at[j]], sems[b], add=True)

    def _sw(j, b):
        pltpu.make_async_copy(rows2.at[b], acc.at[tidx.at[j]],
                              sems[b]).wait()

    def _step(j, b):
        _gw(j, b)
        _ss(j, b)
        _sw(j - 1, 1 - b)
        _gs(j + 1, 1 - b)

    ROUND = MAXCH // 2
    for r in range(2):
        ro = r * ROUND
        pltpu.sync_copy(e2d.at[0, pl.ds(start + ro, ROUND)], fidx)
        pltpu.sync_copy(e2d.at[1, pl.ds(start + ro, ROUND)], tidx)
        _gs(0, 0)
        _gw(0, 0)
        _ss(0, 0)
        _gs(1, 1)

        @pl.loop(0, (ROUND - 2) // 2)
        def _edges(g):
            _step(2 * g + 1, 1)
            _step(2 * g + 2, 0)

        _gw(ROUND - 1, 1)
        _ss(ROUND - 1, 1)
        _sw(ROUND - 2, 0)
        _sw(ROUND - 1, 1)

    plsc.subcore_barrier()

    @pl.when(s < 15)
    def _dump_main():
        pltpu.sync_copy(acc.at[pl.ds(rbase, RPT)],
                        out.at[c, pl.ds(rbase, RPT)])

    @pl.when(s == 15)
    def _dump_tail():
        pltpu.sync_copy(acc.at[pl.ds(rbase, TAILR)],
                        out.at[c, pl.ds(rbase, TAILR)])


_GRID = 10
_R = N // _GRID


def _prep_body(d0_ref, d1_ref, emb_ref, u0_ref, dinv_ref, dinv2_ref):
    deg = d0_ref[...] + d1_ref[...]
    dinv = jnp.where(deg > 0, lax.rsqrt(deg), 0.0)
    dinv_ref[...] = dinv
    dinv2_ref[...] = dinv * dinv
    u0_ref[...] = emb_ref[...] * dinv


_prep = pl.pallas_call(
    _prep_body,
    grid=(_GRID,),
    in_specs=[
        pl.BlockSpec((_R, 1), lambda i: (i, 0)),
        pl.BlockSpec((_R, 1), lambda i: (i, 0)),
        pl.BlockSpec((_R, D), lambda i: (i, 0)),
    ],
    out_specs=[
        pl.BlockSpec((_R, D), lambda i: (i, 0)),
        pl.BlockSpec((_R, 1), lambda i: (i, 0)),
        pl.BlockSpec((_R, 1), lambda i: (i, 0)),
    ],
    out_shape=[
        jax.ShapeDtypeStruct((N, D), jnp.float32),
        jax.ShapeDtypeStruct((N, 1), jnp.float32),
        jax.ShapeDtypeStruct((N, 1), jnp.float32),
    ],
)


def _scale_body(p_ref, d2_ref, s_ref, u_ref):
    tot = p_ref[0] + p_ref[1]
    s_ref[...] = tot
    u_ref[...] = tot * d2_ref[...]


_scale = pl.pallas_call(
    _scale_body,
    grid=(_GRID,),
    in_specs=[
        pl.BlockSpec((2, _R, D), lambda i: (0, i, 0)),
        pl.BlockSpec((_R, 1), lambda i: (i, 0)),
    ],
    out_specs=[
        pl.BlockSpec((_R, D), lambda i: (i, 0)),
        pl.BlockSpec((_R, D), lambda i: (i, 0)),
    ],
    out_shape=[
        jax.ShapeDtypeStruct((N, D), jnp.float32),
        jax.ShapeDtypeStruct((N, D), jnp.float32),
    ],
)


def _final_body(emb_ref, dinv_ref, s1_ref, s2_ref, p3_ref, o_ref):
    tot = s1_ref[...] + s2_ref[...] + p3_ref[0] + p3_ref[1]
    o_ref[...] = 0.25 * (emb_ref[...] + dinv_ref[...] * tot)


_final = pl.pallas_call(
    _final_body,
    grid=(_GRID,),
    in_specs=[
        pl.BlockSpec((_R, D), lambda i: (i, 0)),
        pl.BlockSpec((_R, 1), lambda i: (i, 0)),
        pl.BlockSpec((_R, D), lambda i: (i, 0)),
        pl.BlockSpec((_R, D), lambda i: (i, 0)),
        pl.BlockSpec((2, _R, D), lambda i: (0, i, 0)),
    ],
    out_specs=pl.BlockSpec((_R, D), lambda i: (i, 0)),
    out_shape=jax.ShapeDtypeStruct((N, D), jnp.float32),
)


def kernel(edge_list, emb_weight):
    pad_from = (jnp.arange(PAD, dtype=jnp.int32) * 797) % N
    pad_to = N + (jnp.arange(PAD, dtype=jnp.int32) % 8)
    e2d = jnp.concatenate(
        [edge_list, jnp.stack([pad_from, pad_to])], axis=1,
    ).reshape(2, NCHP, CHUNK)
    d0, d1 = _deg_kernel(e2d)
    u0, dinv, dinv2 = _prep(d0.reshape(N, 1), d1.reshape(N, 1), emb_weight)
    p1 = _hop_kernel(e2d, u0)
    s1, u1 = _scale(p1, dinv2)
    p2 = _hop_kernel(e2d, u1)
    s2, u2 = _scale(p2, dinv2)
    p3 = _hop_kernel(e2d, u2)
    mean = _final(emb_weight, dinv, s1, s2, p3)
    return (emb_weight, mean)


# speedup vs baseline: 1.1088x; 1.1088x over previous
import functools

import jax
import jax.numpy as jnp
from jax import lax
from jax.experimental import pallas as pl
from jax.experimental.pallas import tpu as pltpu
from jax.experimental.pallas import tpu_sc as plsc

N = 10000
D = 128
E = 320000
LAYERS = 3

CHUNK = 64
W = 32
MAXCH = 160
NCHP = W * MAXCH
PAD = NCHP * CHUNK - E
NROWS = N + 8
RPT = 640
TAILR = N - 15 * RPT

_mesh = plsc.VectorSubcoreMesh(core_axis_name="c", subcore_axis_name="s")


@functools.partial(
    pl.kernel,
    out_type=[jax.ShapeDtypeStruct((N,), jnp.float32),
              jax.ShapeDtypeStruct((N,), jnp.float32)],
    mesh=_mesh,
    scratch_types=[
        pltpu.VMEM_SHARED((NROWS,), jnp.float32),
        pltpu.VMEM((MAXCH, CHUNK), jnp.int32),
        pltpu.VMEM((CHUNK,), jnp.float32),
        pltpu.VMEM((640,), jnp.float32),
    ],
)
def _deg_kernel(e2d, dout0, dout1, acc, tidx, ones, z):
    c = lax.axis_index("c")
    s = lax.axis_index("s")
    start = (c * 16 + s) * MAXCH
    for i in range(40):
        z[pl.ds(i * 16, 16)] = jnp.zeros((16,), jnp.float32)
    for i in range(CHUNK // 16):
        ones[pl.ds(i * 16, 16)] = jnp.ones((16,), jnp.float32)
    zbase = pl.multiple_of(s * 624, 8)
    pltpu.sync_copy(z, acc.at[pl.ds(zbase, 640)])
    plsc.subcore_barrier()
    pltpu.sync_copy(e2d.at[1, pl.ds(start, MAXCH)], tidx)

    @pl.loop(0, MAXCH)
    def _scatter(j):
        pltpu.sync_copy(ones, acc.at[tidx.at[j]], add=True)

    plsc.subcore_barrier()

    for ci, dref in ((0, dout0), (1, dout1)):
        @pl.when((c == ci) & (s < 15))
        def _dump_main(dref=dref):
            o = pl.multiple_of(s * 624, 8)
            pltpu.sync_copy(acc.at[pl.ds(o, 624)], z.at[pl.ds(0, 624)])
            pltpu.sync_copy(z.at[pl.ds(0, 624)], dref.at[pl.ds(o, 624)])

        @pl.when((c == ci) & (s == 15))
        def _dump_tail(dref=dref):
            pltpu.sync_copy(acc.at[pl.ds(9360, 640)], z)
            pltpu.sync_copy(z, dref.at[pl.ds(9360, 640)])


@functools.partial(
    pl.kernel,
    out_type=jax.ShapeDtypeStruct((2, N, D), jnp.float32),
    mesh=_mesh,
    scratch_types=[
        pltpu.VMEM_SHARED((NROWS, D), jnp.float32),
        pltpu.VMEM((MAXCH // 4, CHUNK), jnp.int32),
        pltpu.VMEM((MAXCH // 4, CHUNK), jnp.int32),
        pltpu.VMEM((4, CHUNK, D), jnp.float32),
        pltpu.SemaphoreType.DMA,
        pltpu.SemaphoreType.DMA,
        pltpu.SemaphoreType.DMA,
        pltpu.SemaphoreType.DMA,
        pltpu.SemaphoreType.DMA,
        pltpu.SemaphoreType.DMA,
        pltpu.SemaphoreType.DMA,
        pltpu.SemaphoreType.DMA,
    ],
)
def _hop_kernel(e2d, u, out, acc, fidx, tidx, rows2,
                semg0, semg1, semg2, semg3, sems0, sems1, sems2, sems3):
    rows = rows2.at[0]
    semg = (semg0, semg1, semg2, semg3)
    sems = (sems0, sems1, sems2, sems3)
    c = lax.axis_index("c")
    s = lax.axis_index("s")
    start = (c * 16 + s) * MAXCH

    @pl.loop(0, CHUNK)
    def _zrow(r):
        for k in range(D // 16):
            rows[r, pl.ds(k * 16, 16)] = jnp.zeros((16,), jnp.float32)

    rbase = s * RPT

    @pl.when(s < 15)
    def _zero_main():
        for b in range(RPT // CHUNK):
            pltpu.sync_copy(rows, acc.at[pl.ds(rbase + b * CHUNK, CHUNK)])

    @pl.when(s == 15)
    def _zero_tail():
        for b in range(TAILR // CHUNK):
            pltpu.sync_copy(rows, acc.at[pl.ds(rbase + b * CHUNK, CHUNK)])
        pltpu.sync_copy(rows.at[pl.ds(0, TAILR % CHUNK)],
                        acc.at[pl.ds(rbase + (TAILR // CHUNK) * CHUNK,
                                     TAILR % CHUNK)])

    plsc.subcore_barrier()

    def _gs(j, b):
        return pltpu.async_copy(u.at[fidx.at[j]], rows2.at[b], semg[b])

    def _gw(j, b):
        pltpu.make_async_copy(u.at[fidx.at[j]], rows2.at[b], semg[b]).wait()

    def _ss(j, b):
        pltpu.async_copy(rows2.at[b], acc.at[tidx.at[j]], sems[b], add=True)

    def _sw(j, b):
        pltpu.make_async_copy(rows2.at[b], acc.at[tidx.at[j]],
                              sems[b]).wait()

    def _step(j, b):
        _gw(j, b)
        _ss(j, b)
        _sw(j - 1, (b - 1) % 4)
        _gs(j + 3, (b - 1) % 4)

    ROUND = MAXCH // 4
    for r in range(4):
        ro = r * ROUND
        pltpu.sync_copy(e2d.at[0, pl.ds(start + ro, ROUND)], fidx)
        pltpu.sync_copy(e2d.at[1, pl.ds(start + ro, ROUND)], tidx)
        _gs(0, 0)
        _gs(1, 1)
        _gs(2, 2)
        _gw(0, 0)
        _ss(0, 0)
        _gs(3, 3)

        @pl.loop(0, (ROUND - 4) // 4)
        def _edges(g):
            _step(4 * g + 1, 1)
            _step(4 * g + 2, 2)
            _step(4 * g + 3, 3)
            _step(4 * g + 4, 0)

        _gw(ROUND - 3, 1)
        _ss(ROUND - 3, 1)
        _sw(ROUND - 4, 0)
        _gw(ROUND - 2, 2)
        _ss(ROUND - 2, 2)
        _sw(ROUND - 3, 1)
        _gw(ROUND - 1, 3)
        _ss(ROUND - 1, 3)
        _sw(ROUND - 2, 2)
        _sw(ROUND - 1, 3)

    plsc.subcore_barrier()

    @pl.when(s < 15)
    def _dump_main():
        pltpu.sync_copy(acc.at[pl.ds(rbase, RPT)],
                        out.at[c, pl.ds(rbase, RPT)])

    @pl.when(s == 15)
    def _dump_tail():
        pltpu.sync_copy(acc.at[pl.ds(rbase, TAILR)],
                        out.at[c, pl.ds(rbase, TAILR)])


_GRID = 10
_R = N // _GRID


def _prep_body(d0_ref, d1_ref, emb_ref, u0_ref, dinv_ref, dinv2_ref):
    deg = d0_ref[...] + d1_ref[...]
    dinv = jnp.where(deg > 0, lax.rsqrt(deg), 0.0)
    dinv_ref[...] = dinv
    dinv2_ref[...] = dinv * dinv
    u0_ref[...] = emb_ref[...] * dinv


_prep = pl.pallas_call(
    _prep_body,
    grid=(_GRID,),
    in_specs=[
        pl.BlockSpec((_R, 1), lambda i: (i, 0)),
        pl.BlockSpec((_R, 1), lambda i: (i, 0)),
        pl.BlockSpec((_R, D), lambda i: (i, 0)),
    ],
    out_specs=[
        pl.BlockSpec((_R, D), lambda i: (i, 0)),
        pl.BlockSpec((_R, 1), lambda i: (i, 0)),
        pl.BlockSpec((_R, 1), lambda i: (i, 0)),
    ],
    out_shape=[
        jax.ShapeDtypeStruct((N, D), jnp.float32),
        jax.ShapeDtypeStruct((N, 1), jnp.float32),
        jax.ShapeDtypeStruct((N, 1), jnp.float32),
    ],
)


def _scale_body(p_ref, d2_ref, s_ref, u_ref):
    tot = p_ref[0] + p_ref[1]
    s_ref[...] = tot
    u_ref[...] = tot * d2_ref[...]


_scale = pl.pallas_call(
    _scale_body,
    grid=(_GRID,),
    in_specs=[
        pl.BlockSpec((2, _R, D), lambda i: (0, i, 0)),
        pl.BlockSpec((_R, 1), lambda i: (i, 0)),
    ],
    out_specs=[
        pl.BlockSpec((_R, D), lambda i: (i, 0)),
        pl.BlockSpec((_R, D), lambda i: (i, 0)),
    ],
    out_shape=[
        jax.ShapeDtypeStruct((N, D), jnp.float32),
        jax.ShapeDtypeStruct((N, D), jnp.float32),
    ],
)


def _final_body(emb_ref, dinv_ref, s1_ref, s2_ref, p3_ref, o_ref):
    tot = s1_ref[...] + s2_ref[...] + p3_ref[0] + p3_ref[1]
    o_ref[...] = 0.25 * (emb_ref[...] + dinv_ref[...] * tot)


_final = pl.pallas_call(
    _final_body,
    grid=(_GRID,),
    in_specs=[
        pl.BlockSpec((_R, D), lambda i: (i, 0)),
        pl.BlockSpec((_R, 1), lambda i: (i, 0)),
        pl.BlockSpec((_R, D), lambda i: (i, 0)),
        pl.BlockSpec((_R, D), lambda i: (i, 0)),
        pl.BlockSpec((2, _R, D), lambda i: (0, i, 0)),
    ],
    out_specs=pl.BlockSpec((_R, D), lambda i: (i, 0)),
    out_shape=jax.ShapeDtypeStruct((N, D), jnp.float32),
)


def kernel(edge_list, emb_weight):
    pad_from = (jnp.arange(PAD, dtype=jnp.int32) * 797) % N
    pad_to = N + (jnp.arange(PAD, dtype=jnp.int32) % 8)
    e2d = jnp.concatenate(
        [edge_list, jnp.stack([pad_from, pad_to])], axis=1,
    ).reshape(2, NCHP, CHUNK)
    d0, d1 = _deg_kernel(e2d)
    u0, dinv, dinv2 = _prep(d0.reshape(N, 1), d1.reshape(N, 1), emb_weight)
    p1 = _hop_kernel(e2d, u0)
    s1, u1 = _scale(p1, dinv2)
    p2 = _hop_kernel(e2d, u1)
    s2, u2 = _scale(p2, dinv2)
    p3 = _hop_kernel(e2d, u2)
    mean = _final(emb_weight, dinv, s1, s2, p3)
    return (emb_weight, mean)


# speedup vs baseline: 1.1374x; 1.0259x over previous
import functools

import jax
import jax.numpy as jnp
from jax import lax
from jax.experimental import pallas as pl
from jax.experimental.pallas import tpu as pltpu
from jax.experimental.pallas import tpu_sc as plsc

N = 10000
D = 128
E = 320000
LAYERS = 3

CHUNK = 64
W = 32
MAXCH = 160
NCHP = W * MAXCH
PAD = NCHP * CHUNK - E
NROWS = N + 8
RPT = 640
TAILR = N - 15 * RPT

_mesh = plsc.VectorSubcoreMesh(core_axis_name="c", subcore_axis_name="s")


@functools.partial(
    pl.kernel,
    out_type=[jax.ShapeDtypeStruct((N,), jnp.float32),
              jax.ShapeDtypeStruct((N,), jnp.float32)],
    mesh=_mesh,
    scratch_types=[
        pltpu.VMEM_SHARED((NROWS,), jnp.float32),
        pltpu.VMEM((MAXCH, CHUNK), jnp.int32),
        pltpu.VMEM((CHUNK,), jnp.float32),
        pltpu.VMEM((640,), jnp.float32),
    ],
)
def _deg_kernel(e2d, dout0, dout1, acc, tidx, ones, z):
    c = lax.axis_index("c")
    s = lax.axis_index("s")
    start = (c * 16 + s) * MAXCH
    for i in range(40):
        z[pl.ds(i * 16, 16)] = jnp.zeros((16,), jnp.float32)
    for i in range(CHUNK // 16):
        ones[pl.ds(i * 16, 16)] = jnp.ones((16,), jnp.float32)
    zbase = pl.multiple_of(s * 624, 8)
    pltpu.sync_copy(z, acc.at[pl.ds(zbase, 640)])
    plsc.subcore_barrier()
    pltpu.sync_copy(e2d.at[1, pl.ds(start, MAXCH)], tidx)

    @pl.loop(0, MAXCH)
    def _scatter(j):
        pltpu.sync_copy(ones, acc.at[tidx.at[j]], add=True)

    plsc.subcore_barrier()

    for ci, dref in ((0, dout0), (1, dout1)):
        @pl.when((c == ci) & (s < 15))
        def _dump_main(dref=dref):
            o = pl.multiple_of(s * 624, 8)
            pltpu.sync_copy(acc.at[pl.ds(o, 624)], z.at[pl.ds(0, 624)])
            pltpu.sync_copy(z.at[pl.ds(0, 624)], dref.at[pl.ds(o, 624)])

        @pl.when((c == ci) & (s == 15))
        def _dump_tail(dref=dref):
            pltpu.sync_copy(acc.at[pl.ds(9360, 640)], z)
            pltpu.sync_copy(z, dref.at[pl.ds(9360, 640)])


@functools.partial(
    pl.kernel,
    out_type=jax.ShapeDtypeStruct((2, N, D), jnp.float32),
    mesh=_mesh,
    scratch_types=[
        pltpu.VMEM_SHARED((NROWS, D), jnp.float32),
        pltpu.VMEM((MAXCH // 5, CHUNK), jnp.int32),
        pltpu.VMEM((MAXCH // 5, CHUNK), jnp.int32),
        pltpu.VMEM((5, CHUNK, D), jnp.float32),
        pltpu.SemaphoreType.DMA,
        pltpu.SemaphoreType.DMA,
        pltpu.SemaphoreType.DMA,
        pltpu.SemaphoreType.DMA,
        pltpu.SemaphoreType.DMA,
        pltpu.SemaphoreType.DMA,
        pltpu.SemaphoreType.DMA,
        pltpu.SemaphoreType.DMA,
        pltpu.SemaphoreType.DMA,
        pltpu.SemaphoreType.DMA,
    ],
)
def _hop_kernel(e2d, u, out, acc, fidx, tidx, rows2,
                semg0, semg1, semg2, semg3, semg4,
                sems0, sems1, sems2, sems3, sems4):
    rows = rows2.at[0]
    semg = (semg0, semg1, semg2, semg3, semg4)
    sems = (sems0, sems1, sems2, sems3, sems4)
    c = lax.axis_index("c")
    s = lax.axis_index("s")
    start = (c * 16 + s) * MAXCH

    @pl.loop(0, CHUNK)
    def _zrow(r):
        for k in range(D // 16):
            rows[r, pl.ds(k * 16, 16)] = jnp.zeros((16,), jnp.float32)

    rbase = s * RPT

    @pl.when(s < 15)
    def _zero_main():
        for b in range(RPT // CHUNK):
            pltpu.sync_copy(rows, acc.at[pl.ds(rbase + b * CHUNK, CHUNK)])

    @pl.when(s == 15)
    def _zero_tail():
        for b in range(TAILR // CHUNK):
            pltpu.sync_copy(rows, acc.at[pl.ds(rbase + b * CHUNK, CHUNK)])
        pltpu.sync_copy(rows.at[pl.ds(0, TAILR % CHUNK)],
                        acc.at[pl.ds(rbase + (TAILR // CHUNK) * CHUNK,
                                     TAILR % CHUNK)])

    plsc.subcore_barrier()

    def _gs(j, b):
        return pltpu.async_copy(u.at[fidx.at[j]], rows2.at[b], semg[b])

    def _gw(j, b):
        pltpu.make_async_copy(u.at[fidx.at[j]], rows2.at[b], semg[b]).wait()

    def _ss(j, b):
        pltpu.async_copy(rows2.at[b], acc.at[tidx.at[j]], sems[b], add=True)

    def _sw(j, b):
        pltpu.make_async_copy(rows2.at[b], acc.at[tidx.at[j]],
                              sems[b]).wait()

    def _step(j, b):
        _gw(j, b)
        _ss(j, b)
        _sw(j - 1, (b - 1) % 5)
        _gs(j + 4, (b - 1) % 5)

    ROUND = MAXCH // 5
    for r in range(5):
        ro = r * ROUND
        pltpu.sync_copy(e2d.at[0, pl.ds(start + ro, ROUND)], fidx)
        pltpu.sync_copy(e2d.at[1, pl.ds(start + ro, ROUND)], tidx)
        _gs(0, 0)
        _gs(1, 1)
        _gs(2, 2)
        _gs(3, 3)
        _gw(0, 0)
        _ss(0, 0)
        _gs(4, 4)

        @pl.loop(0, (ROUND - 7) // 5)
        def _edges(g):
            _step(5 * g + 1, 1)
            _step(5 * g + 2, 2)
            _step(5 * g + 3, 3)
            _step(5 * g + 4, 4)
            _step(5 * g + 5, 0)

        _step(ROUND - 6, 1)
        _step(ROUND - 5, 2)
        _gw(ROUND - 4, 3)
        _ss(ROUND - 4, 3)
        _sw(ROUND - 5, 2)
        _gw(ROUND - 3, 4)
        _ss(ROUND - 3, 4)
        _sw(ROUND - 4, 3)
        _gw(ROUND - 2, 0)
        _ss(ROUND - 2, 0)
        _sw(ROUND - 3, 4)
        _gw(ROUND - 1, 1)
        _ss(ROUND - 1, 1)
        _sw(ROUND - 2, 0)
        _sw(ROUND - 1, 1)

    plsc.subcore_barrier()

    @pl.when(s < 15)
    def _dump_main():
        pltpu.sync_copy(acc.at[pl.ds(rbase, RPT)],
                        out.at[c, pl.ds(rbase, RPT)])

    @pl.when(s == 15)
    def _dump_tail():
        pltpu.sync_copy(acc.at[pl.ds(rbase, TAILR)],
                        out.at[c, pl.ds(rbase, TAILR)])


_GRID = 10
_R = N // _GRID


def _prep_body(d0_ref, d1_ref, emb_ref, u0_ref, dinv_ref, dinv2_ref):
    deg = d0_ref[...] + d1_ref[...]
    dinv = jnp.where(deg > 0, lax.rsqrt(deg), 0.0)
    dinv_ref[...] = dinv
    dinv2_ref[...] = dinv * dinv
    u0_ref[...] = emb_ref[...] * dinv


_prep = pl.pallas_call(
    _prep_body,
    grid=(_GRID,),
    in_specs=[
        pl.BlockSpec((_R, 1), lambda i: (i, 0)),
        pl.BlockSpec((_R, 1), lambda i: (i, 0)),
        pl.BlockSpec((_R, D), lambda i: (i, 0)),
    ],
    out_specs=[
        pl.BlockSpec((_R, D), lambda i: (i, 0)),
        pl.BlockSpec((_R, 1), lambda i: (i, 0)),
        pl.BlockSpec((_R, 1), lambda i: (i, 0)),
    ],
    out_shape=[
        jax.ShapeDtypeStruct((N, D), jnp.float32),
        jax.ShapeDtypeStruct((N, 1), jnp.float32),
        jax.ShapeDtypeStruct((N, 1), jnp.float32),
    ],
)


def _scale_body(p_ref, d2_ref, s_ref, u_ref):
    tot = p_ref[0] + p_ref[1]
    s_ref[...] = tot
    u_ref[...] = tot * d2_ref[...]


_scale = pl.pallas_call(
    _scale_body,
    grid=(_GRID,),
    in_specs=[
        pl.BlockSpec((2, _R, D), lambda i: (0, i, 0)),
        pl.BlockSpec((_R, 1), lambda i: (i, 0)),
    ],
    out_specs=[
        pl.BlockSpec((_R, D), lambda i: (i, 0)),
        pl.BlockSpec((_R, D), lambda i: (i, 0)),
    ],
    out_shape=[
        jax.ShapeDtypeStruct((N, D), jnp.float32),
        jax.ShapeDtypeStruct((N, D), jnp.float32),
    ],
)


def _final_body(emb_ref, dinv_ref, s1_ref, s2_ref, p3_ref, o_ref):
    tot = s1_ref[...] + s2_ref[...] + p3_ref[0] + p3_ref[1]
    o_ref[...] = 0.25 * (emb_ref[...] + dinv_ref[...] * tot)


_final = pl.pallas_call(
    _final_body,
    grid=(_GRID,),
    in_specs=[
        pl.BlockSpec((_R, D), lambda i: (i, 0)),
        pl.BlockSpec((_R, 1), lambda i: (i, 0)),
        pl.BlockSpec((_R, D), lambda i: (i, 0)),
        pl.BlockSpec((_R, D), lambda i: (i, 0)),
        pl.BlockSpec((2, _R, D), lambda i: (0, i, 0)),
    ],
    out_specs=pl.BlockSpec((_R, D), lambda i: (i, 0)),
    out_shape=jax.ShapeDtypeStruct((N, D), jnp.float32),
)


def kernel(edge_list, emb_weight):
    pad_from = (jnp.arange(PAD, dtype=jnp.int32) * 797) % N
    pad_to = N + (jnp.arange(PAD, dtype=jnp.int32) % 8)
    e2d = jnp.concatenate(
        [edge_list, jnp.stack([pad_from, pad_to])], axis=1,
    ).reshape(2, NCHP, CHUNK)
    d0, d1 = _deg_kernel(e2d)
    u0, dinv, dinv2 = _prep(d0.reshape(N, 1), d1.reshape(N, 1), emb_weight)
    p1 = _hop_kernel(e2d, u0)
    s1, u1 = _scale(p1, dinv2)
    p2 = _hop_kernel(e2d, u1)
    s2, u2 = _scale(p2, dinv2)
    p3 = _hop_kernel(e2d, u2)
    mean = _final(emb_weight, dinv, s1, s2, p3)
    return (emb_weight, mean)
